# baseline (speedup 1.0000x reference)
import numpy as np
import jax
import jax.numpy as jnp
from jax.experimental import pallas as pl
from jax.experimental.pallas import tpu as pltpu

_T = np.float32(1.3)
_QHAT = np.float32(0.93)
_KREG = 5
_LAMDA = np.float32(0.01)
_K = 98
_KPAD = 128
_RB = 32
_NREG = 98
_RW = 1024
_CPAD = _NREG * _RW
_NSWEEP = 10
_BIG = np.int32(_CPAD)
_NEG = np.float32(-1.0)


def _body(x_hbm, mask_ref, x2_ref, sw_ref, vals_ref, idxs_ref, dma_sem):
    rb, c = x2_ref.shape
    i = pl.program_id(0)
    cp = pltpu.make_async_copy(x_hbm.at[pl.ds(i * rb, rb), :], x2_ref, dma_sem)
    cp.start()
    cp.wait()
    xt = x2_ref[...] / _T
    m = jnp.max(xt, axis=1, keepdims=True)
    e = jnp.exp(xt - m)
    z = jnp.sum(e, axis=1, keepdims=True)
    sw_ref[...] = jnp.full((rb, _CPAD), _NEG)
    sw_ref[:, :c] = e / z

    riota = jax.lax.broadcasted_iota(jnp.int32, (rb, _RW), 1)
    liota = jax.lax.broadcasted_iota(jnp.int32, (rb, _KPAD), 1)
    piota = jax.lax.broadcasted_iota(jnp.int32, (rb, _NSWEEP * _KPAD), 1)

    def sweep(t, pool):
        poolv, pooli = pool
        cv = jnp.full((rb, _KPAD), _NEG)
        ci = jnp.full((rb, _KPAD), _BIG)
        for j in range(_NREG):
            reg = sw_ref[:, j * _RW:(j + 1) * _RW]
            v = jnp.max(reg, axis=1, keepdims=True)
            floc = jnp.argmax(reg, axis=1).astype(jnp.int32)[:, None]
            sw_ref[:, j * _RW:(j + 1) * _RW] = jnp.where(riota == floc,
                                                         _NEG, reg)
            cv = jnp.where(liota == j, v, cv)
            ci = jnp.where(liota == j, floc + j * _RW, ci)
        slot = (piota >= t * _KPAD) & (piota < (t + 1) * _KPAD)
        poolv = jnp.where(slot, jnp.concatenate([cv] * _NSWEEP, axis=1), poolv)
        pooli = jnp.where(slot, jnp.concatenate([ci] * _NSWEEP, axis=1), pooli)
        return poolv, pooli

    poolv0 = jnp.full((rb, _NSWEEP * _KPAD), _NEG)
    pooli0 = jnp.full((rb, _NSWEEP * _KPAD), _BIG)
    poolv, pooli = jax.lax.fori_loop(0, _NSWEEP, sweep, (poolv0, pooli0))

    def merge_step(k, carry):
        pv, pi, vals, idxs = carry
        v = jnp.max(pv, axis=1, keepdims=True)
        fi = jnp.min(jnp.where(pv == v, pi, _BIG), axis=1, keepdims=True)
        pv = jnp.where((pv == v) & (pi == fi), _NEG, pv)
        vals = jnp.where(liota == k, v, vals)
        idxs = jnp.where(liota == k, fi, idxs)
        return pv, pi, vals, idxs

    zero_v = jnp.zeros((rb, _KPAD), jnp.float32)
    zero_i = jnp.zeros((rb, _KPAD), jnp.int32)
    _, _, vals, idxs = jax.lax.fori_loop(
        0, _K, merge_step, (poolv, pooli, zero_v, zero_i))
    vals_ref[...] = vals
    idxs_ref[...] = idxs

    lv = poolv[:, (_NSWEEP - 1) * _KPAD:_NSWEEP * _KPAD]
    li = pooli[:, (_NSWEEP - 1) * _KPAD:_NSWEEP * _KPAD]
    tv = vals[:, _K - 1:_K]
    ti = idxs[:, _K - 1:_K]
    reg_ok = (lv < tv) | ((lv == tv) & (li >= ti))
    reg_ok = reg_ok | (liota >= _NREG)
    cert = jnp.min(reg_ok.astype(jnp.int32), axis=1, keepdims=True)
    allcert = jnp.min(cert)

    @pl.when(allcert == 0)
    def _fallback():
        sw_ref[:, :c] = jnp.exp(x2_ref[:, :c] / _T - m) / z
        fiota = jax.lax.broadcasted_iota(jnp.int32, (rb, _CPAD), 1)

        def slow_step(k, carry):
            svals, sidxs = carry
            sw = sw_ref[...]
            v = jnp.max(sw, axis=1, keepdims=True)
            fi = jnp.min(jnp.where(sw == v, fiota, _BIG),
                         axis=1, keepdims=True)
            sw_ref[...] = jnp.where(fiota == fi, _NEG, sw)
            svals = jnp.where(liota == k, v, svals)
            sidxs = jnp.where(liota == k, fi, sidxs)
            return svals, sidxs

        svals, sidxs = jax.lax.fori_loop(0, _K, slow_step, (zero_v, zero_i))
        certb = cert > 0
        vals_ref[...] = jnp.where(certb, vals_ref[...], svals)
        idxs_ref[...] = jnp.where(certb, idxs_ref[...], sidxs)

    vals = vals_ref[...]
    idxs = idxs_ref[...]
    ti2 = jax.lax.broadcasted_iota(jnp.int32, (_KPAD, _KPAD), 0)
    tj2 = jax.lax.broadcasted_iota(jnp.int32, (_KPAD, _KPAD), 1)
    tri = (ti2 <= tj2).astype(jnp.float32)
    cs = jax.lax.dot_general(vals, tri, (((1,), (0,)), ((), ())),
                             preferred_element_type=jnp.float32)
    pen = _LAMDA * jnp.maximum(liota - (_KREG - 1), 0).astype(jnp.float32)
    cond = (cs + pen) <= _QHAT
    sizes = jnp.sum(cond.astype(jnp.int32), axis=1, keepdims=True) + 1
    sel_last = liota == (sizes - 1)
    lam = jnp.sum(jnp.where(sel_last, vals, np.float32(0.0)),
                  axis=1, keepdims=True)
    ilast = jnp.sum(jnp.where(sel_last, idxs, 0), axis=1, keepdims=True)

    ciota = jax.lax.broadcasted_iota(jnp.int32, (rb, c), 1)
    s0 = jnp.exp(x2_ref[...] / _T - m) / z
    mask = (s0 > lam) | ((s0 == lam) & (ciota <= ilast))
    mask_ref[...] = mask.astype(jnp.int8)


def kernel(logits):
    b, c = logits.shape
    mask8 = pl.pallas_call(
        _body,
        grid=(b // _RB,),
        in_specs=[pl.BlockSpec(memory_space=pl.ANY)],
        out_specs=pl.BlockSpec((_RB, c), lambda i: (i, 0)),
        out_shape=jax.ShapeDtypeStruct((b, c), jnp.int8),
        scratch_shapes=[pltpu.VMEM((_RB, c), jnp.float32),
                        pltpu.VMEM((_RB, _CPAD), jnp.float32),
                        pltpu.VMEM((_RB, _KPAD), jnp.float32),
                        pltpu.VMEM((_RB, _KPAD), jnp.int32),
                        pltpu.SemaphoreType.DMA],
    )(logits)
    return (logits, mask8.astype(bool))


# speedup vs baseline: 1.7954x; 1.7954x over previous
import numpy as np
import jax
import jax.numpy as jnp
from jax.experimental import pallas as pl
from jax.experimental.pallas import tpu as pltpu

_T = np.float32(1.3)
_QHAT = np.float32(0.93)
_KREG = 5
_LAMDA = np.float32(0.01)
_K = 98
_KPAD = 128
_RB = 32
_NREG = 98
_RW = 1024
_CPAD = _NREG * _RW
_NSWEEP = 10
_BIG = np.int32(_CPAD)
_NEG = np.float32(-1.0)


def _body(x_hbm, mask_ref, x2_ref, sw_ref, vals_ref, idxs_ref, dma_sem):
    rb, c = x2_ref.shape
    i = pl.program_id(0)
    cp = pltpu.make_async_copy(x_hbm.at[pl.ds(i * rb, rb), :], x2_ref, dma_sem)
    cp.start()
    cp.wait()
    xt = x2_ref[...] / _T
    m = jnp.max(xt, axis=1, keepdims=True)
    e = jnp.exp(xt - m)
    z = jnp.sum(e, axis=1, keepdims=True)
    sw_ref[...] = jnp.full((rb, _CPAD), _NEG)
    sw_ref[:, :c] = e / z

    riota = jax.lax.broadcasted_iota(jnp.int32, (rb, _RW), 1)
    liota = jax.lax.broadcasted_iota(jnp.int32, (rb, _KPAD), 1)
    piota = jax.lax.broadcasted_iota(jnp.int32, (rb, _NSWEEP * _KPAD), 1)

    def sweep_cond(carry):
        t, done, _, _ = carry
        return (t < _NSWEEP) & (done == 0)

    def sweep(carry):
        t, _, poolv, pooli = carry
        cv = jnp.full((rb, _KPAD), _NEG)
        ci = jnp.full((rb, _KPAD), _BIG)
        for j in range(_NREG):
            reg = sw_ref[:, j * _RW:(j + 1) * _RW]
            v = jnp.max(reg, axis=1, keepdims=True)
            floc = jnp.min(jnp.where(reg == v, riota, _BIG),
                           axis=1, keepdims=True)
            sw_ref[:, j * _RW:(j + 1) * _RW] = jnp.where(riota == floc,
                                                         _NEG, reg)
            cv = jnp.where(liota == j, v, cv)
            ci = jnp.where(liota == j, floc + j * _RW, ci)
        slot = (piota >= t * _KPAD) & (piota < (t + 1) * _KPAD)
        poolv = jnp.where(slot, jnp.concatenate([cv] * _NSWEEP, axis=1), poolv)
        pooli = jnp.where(slot, jnp.concatenate([ci] * _NSWEEP, axis=1), pooli)
        f = jnp.max(cv, axis=1, keepdims=True)
        cnt = jnp.sum((poolv > f).astype(jnp.int32), axis=1, keepdims=True)
        done = jnp.min(cnt) >= _K
        return t + 1, done.astype(jnp.int32), poolv, pooli

    poolv0 = jnp.full((rb, _NSWEEP * _KPAD), _NEG)
    pooli0 = jnp.full((rb, _NSWEEP * _KPAD), _BIG)
    _, _, poolv, pooli = jax.lax.while_loop(
        sweep_cond, sweep, (jnp.int32(0), jnp.int32(0), poolv0, pooli0))

    def merge_step(k, carry):
        pv, pi, vals, idxs = carry
        v = jnp.max(pv, axis=1, keepdims=True)
        fi = jnp.min(jnp.where(pv == v, pi, _BIG), axis=1, keepdims=True)
        pv = jnp.where((pv == v) & (pi == fi), _NEG, pv)
        vals = jnp.where(liota == k, v, vals)
        idxs = jnp.where(liota == k, fi, idxs)
        return pv, pi, vals, idxs

    zero_v = jnp.zeros((rb, _KPAD), jnp.float32)
    zero_i = jnp.zeros((rb, _KPAD), jnp.int32)
    _, _, vals, idxs = jax.lax.fori_loop(
        0, _K, merge_step, (poolv, pooli, zero_v, zero_i))
    vals_ref[...] = vals
    idxs_ref[...] = idxs

    lv = poolv[:, (_NSWEEP - 1) * _KPAD:_NSWEEP * _KPAD]
    li = pooli[:, (_NSWEEP - 1) * _KPAD:_NSWEEP * _KPAD]
    tv = vals[:, _K - 1:_K]
    ti = idxs[:, _K - 1:_K]
    reg_ok = (lv < tv) | ((lv == tv) & (li >= ti))
    reg_ok = reg_ok | (liota >= _NREG)
    cert = jnp.min(reg_ok.astype(jnp.int32), axis=1, keepdims=True)
    allcert = jnp.min(cert)

    @pl.when(allcert == 0)
    def _fallback():
        sw_ref[:, :c] = jnp.exp(x2_ref[:, :c] / _T - m) / z
        fiota = jax.lax.broadcasted_iota(jnp.int32, (rb, _CPAD), 1)

        def slow_step(k, carry):
            svals, sidxs = carry
            sw = sw_ref[...]
            v = jnp.max(sw, axis=1, keepdims=True)
            fi = jnp.min(jnp.where(sw == v, fiota, _BIG),
                         axis=1, keepdims=True)
            sw_ref[...] = jnp.where(fiota == fi, _NEG, sw)
            svals = jnp.where(liota == k, v, svals)
            sidxs = jnp.where(liota == k, fi, sidxs)
            return svals, sidxs

        svals, sidxs = jax.lax.fori_loop(0, _K, slow_step, (zero_v, zero_i))
        certb = cert > 0
        vals_ref[...] = jnp.where(certb, vals_ref[...], svals)
        idxs_ref[...] = jnp.where(certb, idxs_ref[...], sidxs)

    vals = vals_ref[...]
    idxs = idxs_ref[...]
    ti2 = jax.lax.broadcasted_iota(jnp.int32, (_KPAD, _KPAD), 0)
    tj2 = jax.lax.broadcasted_iota(jnp.int32, (_KPAD, _KPAD), 1)
    tri = (ti2 <= tj2).astype(jnp.float32)
    cs = jax.lax.dot_general(vals, tri, (((1,), (0,)), ((), ())),
                             preferred_element_type=jnp.float32)
    pen = _LAMDA * jnp.maximum(liota - (_KREG - 1), 0).astype(jnp.float32)
    cond = (cs + pen) <= _QHAT
    sizes = jnp.sum(cond.astype(jnp.int32), axis=1, keepdims=True) + 1
    sel_last = liota == (sizes - 1)
    lam = jnp.sum(jnp.where(sel_last, vals, np.float32(0.0)),
                  axis=1, keepdims=True)
    ilast = jnp.sum(jnp.where(sel_last, idxs, 0), axis=1, keepdims=True)

    ciota = jax.lax.broadcasted_iota(jnp.int32, (rb, c), 1)
    s0 = jnp.exp(x2_ref[...] / _T - m) / z
    mask = (s0 > lam) | ((s0 == lam) & (ciota <= ilast))
    mask_ref[...] = mask.astype(jnp.int8)


def kernel(logits):
    b, c = logits.shape
    mask8 = pl.pallas_call(
        _body,
        grid=(b // _RB,),
        in_specs=[pl.BlockSpec(memory_space=pl.ANY)],
        out_specs=pl.BlockSpec((_RB, c), lambda i: (i, 0)),
        out_shape=jax.ShapeDtypeStruct((b, c), jnp.int8),
        scratch_shapes=[pltpu.VMEM((_RB, c), jnp.float32),
                        pltpu.VMEM((_RB, _CPAD), jnp.float32),
                        pltpu.VMEM((_RB, _KPAD), jnp.float32),
                        pltpu.VMEM((_RB, _KPAD), jnp.int32),
                        pltpu.SemaphoreType.DMA],
    )(logits)
    return (logits, mask8.astype(bool))
